# baseline (speedup 1.0000x reference)
import functools

import jax
import jax.numpy as jnp
from jax import lax
from jax.experimental import pallas as pl
from jax.experimental.pallas import tpu as pltpu
from jax.experimental.pallas import tpu_sc as plsc

_TEMP = 0.07


def _tower_body(title_ref, feat_ref, w1t_ref, w1f_ref, b1_ref, w2_ref, b2_ref,
                w3_ref, b3_ref, out_ref):
    h = jnp.dot(title_ref[...], w1t_ref[...], preferred_element_type=jnp.float32)
    h = h + jnp.dot(feat_ref[...], w1f_ref[...], preferred_element_type=jnp.float32)
    h = jax.nn.relu(h + b1_ref[...][None, :])
    h = jax.nn.relu(jnp.dot(h, w2_ref[...], preferred_element_type=jnp.float32)
                    + b2_ref[...][None, :])
    e = jnp.dot(h, w3_ref[...], preferred_element_type=jnp.float32) + b3_ref[...][None, :]
    n = jnp.sqrt(jnp.sum(e * e, axis=-1, keepdims=True))
    out_ref[...] = e / jnp.maximum(n, 1e-12)


def _table_tower(title_emb, item_feat, W1, b1, W2, b2, W3, b3, blk):
    v, title_d = title_emb.shape
    feat_d = item_feat.shape[1]
    d_out = W3.shape[1]
    assert v % blk == 0
    grid = v // blk
    w1t = W1[:title_d]
    w1f = W1[title_d:]
    return pl.pallas_call(
        _tower_body,
        grid=(grid,),
        in_specs=[
            pl.BlockSpec((blk, title_d), lambda i: (i, 0)),
            pl.BlockSpec((blk, feat_d), lambda i: (i, 0)),
            pl.BlockSpec(w1t.shape, lambda i: (0, 0)),
            pl.BlockSpec(w1f.shape, lambda i: (0, 0)),
            pl.BlockSpec(b1.shape, lambda i: (0,)),
            pl.BlockSpec(W2.shape, lambda i: (0, 0)),
            pl.BlockSpec(b2.shape, lambda i: (0,)),
            pl.BlockSpec(W3.shape, lambda i: (0, 0)),
            pl.BlockSpec(b3.shape, lambda i: (0,)),
        ],
        out_specs=pl.BlockSpec((blk, d_out), lambda i: (i, 0)),
        out_shape=jax.ShapeDtypeStruct((v, d_out), jnp.float32),
    )(title_emb, item_feat, w1t, w1f, b1, W2, b2, W3, b3)


def _sc_gather(table, ids, num_cores, num_subcores, chunk):
    n_ids = ids.shape[0]
    d = table.shape[1]
    nw = num_cores * num_subcores
    assert n_ids % (nw * chunk) == 0
    per_w = n_ids // nw
    n_chunks = per_w // chunk

    def body(table_hbm, ids_hbm, out_hbm, idx_v, rows_v, sem):
        wid = lax.axis_index("s") * num_cores + lax.axis_index("c")
        base = wid * per_w
        for k in range(n_chunks):
            off = base + k * chunk
            pltpu.sync_copy(ids_hbm.at[pl.ds(off, chunk)], idx_v)
            pltpu.async_copy(table_hbm.at[idx_v], rows_v, sem).wait()
            pltpu.sync_copy(rows_v, out_hbm.at[pl.ds(off, chunk)])

    return pl.kernel(
        body,
        out_type=jax.ShapeDtypeStruct((n_ids, d), jnp.float32),
        mesh=plsc.VectorSubcoreMesh(core_axis_name="c", subcore_axis_name="s",
                                    num_cores=num_cores,
                                    num_subcores=num_subcores),
        scratch_types=[
            pltpu.VMEM((chunk,), jnp.int32),
            pltpu.VMEM((chunk, d), jnp.float32),
            pltpu.SemaphoreType.DMA,
        ],
        compiler_params=pltpu.CompilerParams(use_tc_tiling_on_sc=False),
    )(table, ids)


def _head_body(gh_ref, gp_ref, r_ref, m_ref, u1_ref, ub1_ref, u2_ref, ub2_ref,
               out_ref):
    w = r_ref[...] * m_ref[...]
    s = jnp.sum(w, axis=1, keepdims=True) + 1e-8
    wn = w / s
    pooled = jnp.sum(wn[:, :, None] * gh_ref[...], axis=1)
    h = jax.nn.relu(jnp.dot(pooled, u1_ref[...], preferred_element_type=jnp.float32)
                    + ub1_ref[...][None, :])
    user = jnp.dot(h, u2_ref[...], preferred_element_type=jnp.float32) + ub2_ref[...][None, :]
    n = jnp.sqrt(jnp.sum(user * user, axis=-1, keepdims=True))
    user = user / jnp.maximum(n, 1e-12)
    out_ref[...] = lax.dot_general(
        user, gp_ref[...], (((1,), (1,)), ((), ())),
        preferred_element_type=jnp.float32) / _TEMP


def _head(g_hist, g_pos, ratings, mask, U1, ub1, U2, ub2, blk):
    bsz, hlen, d = g_hist.shape
    assert bsz % blk == 0
    grid = bsz // blk
    return pl.pallas_call(
        _head_body,
        grid=(grid,),
        in_specs=[
            pl.BlockSpec((blk, hlen, d), lambda i: (i, 0, 0)),
            pl.BlockSpec((bsz, d), lambda i: (0, 0)),
            pl.BlockSpec((blk, hlen), lambda i: (i, 0)),
            pl.BlockSpec((blk, hlen), lambda i: (i, 0)),
            pl.BlockSpec(U1.shape, lambda i: (0, 0)),
            pl.BlockSpec(ub1.shape, lambda i: (0,)),
            pl.BlockSpec(U2.shape, lambda i: (0, 0)),
            pl.BlockSpec(ub2.shape, lambda i: (0,)),
        ],
        out_specs=pl.BlockSpec((blk, bsz), lambda i: (i, 0)),
        out_shape=jax.ShapeDtypeStruct((bsz, bsz), jnp.float32),
    )(g_hist, g_pos, ratings, mask, U1, ub1, U2, ub2)


def kernel(history_items, history_mask, history_ratings, pos_item, title_emb,
           item_feat, W1, b1, W2, b2, W3, b3, U1, ub1, U2, ub2):
    bsz, hlen = history_items.shape
    d_out = W3.shape[1]

    info = plsc.get_sparse_core_info()
    num_cores, num_subcores = info.num_cores, info.num_subcores

    E = _table_tower(title_emb, item_feat, W1, b1, W2, b2, W3, b3, blk=1000)

    ids = jnp.concatenate(
        [history_items.reshape(-1), pos_item]).astype(jnp.int32)
    G = _sc_gather(E, ids, num_cores, num_subcores, chunk=1632)
    return G


# speedup vs baseline: 1.0729x; 1.0729x over previous
import jax
import jax.numpy as jnp
from jax import lax
from jax.experimental import pallas as pl
from jax.experimental.pallas import tpu as pltpu
from jax.experimental.pallas import tpu_sc as plsc

_TEMP = 0.07
_LANES = 16


def _tower_body(title_ref, feat_ref, w1t_ref, w1f_ref, b1_ref, w2_ref, b2_ref,
                w3_ref, b3_ref, out_ref):
    h = jnp.dot(title_ref[...], w1t_ref[...], preferred_element_type=jnp.float32)
    h = h + jnp.dot(feat_ref[...], w1f_ref[...], preferred_element_type=jnp.float32)
    h = jax.nn.relu(h + b1_ref[...][None, :])
    h = jax.nn.relu(jnp.dot(h, w2_ref[...], preferred_element_type=jnp.float32)
                    + b2_ref[...][None, :])
    e = jnp.dot(h, w3_ref[...], preferred_element_type=jnp.float32) + b3_ref[...][None, :]
    n = jnp.sqrt(jnp.sum(e * e, axis=-1, keepdims=True))
    e = e / jnp.maximum(n, 1e-12)
    out_ref[...] = jnp.concatenate([e, jnp.zeros_like(e)], axis=-1)


def _table_tower(title_emb, item_feat, W1, b1, W2, b2, W3, b3, blk):
    v, title_d = title_emb.shape
    feat_d = item_feat.shape[1]
    d_out = W3.shape[1]
    assert v % blk == 0
    grid = v // blk
    w1t = W1[:title_d]
    w1f = W1[title_d:]
    return pl.pallas_call(
        _tower_body,
        grid=(grid,),
        in_specs=[
            pl.BlockSpec((blk, title_d), lambda i: (i, 0)),
            pl.BlockSpec((blk, feat_d), lambda i: (i, 0)),
            pl.BlockSpec(w1t.shape, lambda i: (0, 0)),
            pl.BlockSpec(w1f.shape, lambda i: (0, 0)),
            pl.BlockSpec(b1.shape, lambda i: (0,)),
            pl.BlockSpec(W2.shape, lambda i: (0, 0)),
            pl.BlockSpec(b2.shape, lambda i: (0,)),
            pl.BlockSpec(W3.shape, lambda i: (0, 0)),
            pl.BlockSpec(b3.shape, lambda i: (0,)),
        ],
        out_specs=pl.BlockSpec((blk, 2 * d_out), lambda i: (i, 0)),
        out_shape=jax.ShapeDtypeStruct((v, 2 * d_out), jnp.float32),
    )(title_emb, item_feat, w1t, w1f, b1, W2, b2, W3, b3)


def _splat(vec, lane):
    dnums = lax.GatherDimensionNumbers(
        offset_dims=(), collapsed_slice_dims=(0,), start_index_map=(0,))
    idx = jnp.full((_LANES, 1), lane, jnp.int32)
    return lax.gather(vec, idx, dnums, (1,),
                      mode=lax.GatherScatterMode.PROMISE_IN_BOUNDS)


def _sc_pool(table2, hist_ids, ratings_flat, mask_flat, pos_ids,
             num_cores, num_subcores, rows_per_chunk):
    bl = hist_ids.shape[0]
    b = pos_ids.shape[0]
    hlen = bl // b
    nw = num_cores * num_subcores
    assert b % nw == 0
    b_per_w = b // nw
    rc = rows_per_chunk
    assert b_per_w % rc == 0
    n_chunks = b_per_w // rc
    wpc = rc * hlen
    assert wpc % _LANES == 0
    nwvec = wpc // _LANES
    assert (rc * hlen) % 8 == 0 and b_per_w % 8 == 0

    def body(table_hbm, ids_hbm, r_hbm, m_hbm, pos_hbm, pool_out, pos_out,
             idx_v, rows_v, rbuf, mbuf, pool_v, posi_v, posr_v, sem):
        wid = lax.axis_index("s") * num_cores + lax.axis_index("c")
        row0 = wid * b_per_w
        pltpu.sync_copy(pos_hbm.at[pl.ds(row0, b_per_w)], posi_v)
        pltpu.async_copy(table_hbm.at[posi_v], posr_v, sem).wait()
        pltpu.sync_copy(posr_v, pos_out.at[pl.ds(row0, b_per_w)])

        def chunk_body(c, carry):
            b0 = row0 + c * rc
            pltpu.sync_copy(ids_hbm.at[pl.ds(b0 * hlen, wpc)], idx_v)
            pltpu.async_copy(table_hbm.at[idx_v], rows_v, sem).wait()
            pltpu.sync_copy(r_hbm.at[pl.ds(b0 * hlen, wpc)], rbuf)
            pltpu.sync_copy(m_hbm.at[pl.ds(b0 * hlen, wpc)], mbuf)
            wvecs = [rbuf[pl.ds(k * _LANES, _LANES)] * mbuf[pl.ds(k * _LANES, _LANES)]
                     for k in range(nwvec)]
            for j in range(rc):
                acc = [jnp.zeros((_LANES,), jnp.float32) for _ in range(4)]
                for l in range(hlen):
                    g = j * hlen + l
                    ws = _splat(wvecs[g // _LANES], g % _LANES)
                    for m in range(4):
                        acc[m] = acc[m] + ws * rows_v[g, pl.ds(m * _LANES, _LANES)]
                for m in range(4):
                    pool_v[pl.ds(j * 64 + m * _LANES, _LANES)] = acc[m]
            pltpu.sync_copy(pool_v, pool_out.at[pl.ds(b0 * 64, rc * 64)])
            return carry

        lax.fori_loop(0, n_chunks, chunk_body, 0)

    return pl.kernel(
        body,
        out_type=(
            jax.ShapeDtypeStruct((b * 64,), jnp.float32),
            jax.ShapeDtypeStruct((b, 128), jnp.float32),
        ),
        mesh=plsc.VectorSubcoreMesh(core_axis_name="c", subcore_axis_name="s",
                                    num_cores=num_cores,
                                    num_subcores=num_subcores),
        scratch_types=[
            pltpu.VMEM((wpc,), jnp.int32),
            pltpu.VMEM((wpc, 128), jnp.float32),
            pltpu.VMEM((wpc,), jnp.float32),
            pltpu.VMEM((wpc,), jnp.float32),
            pltpu.VMEM((rc * 64,), jnp.float32),
            pltpu.VMEM((b_per_w,), jnp.int32),
            pltpu.VMEM((b_per_w, 128), jnp.float32),
            pltpu.SemaphoreType.DMA,
        ],
    )(table2, hist_ids, ratings_flat, mask_flat, pos_ids)


def _head_body(pooled_ref, pos_ref, r_ref, m_ref, u1_ref, ub1_ref, u2_ref,
               ub2_ref, out_ref):
    w = r_ref[...] * m_ref[...]
    s = jnp.sum(w, axis=1, keepdims=True) + 1e-8
    pooled = pooled_ref[...] / s
    h = jax.nn.relu(jnp.dot(pooled, u1_ref[...], preferred_element_type=jnp.float32)
                    + ub1_ref[...][None, :])
    user = jnp.dot(h, u2_ref[...], preferred_element_type=jnp.float32) + ub2_ref[...][None, :]
    n = jnp.sqrt(jnp.sum(user * user, axis=-1, keepdims=True))
    user = user / jnp.maximum(n, 1e-12)
    pos = pos_ref[...][:, :64]
    out_ref[...] = lax.dot_general(
        user, pos, (((1,), (1,)), ((), ())),
        preferred_element_type=jnp.float32) / _TEMP


def _head(pooled, pos, ratings, mask, U1, ub1, U2, ub2, blk):
    bsz, d = pooled.shape
    hlen = ratings.shape[1]
    assert bsz % blk == 0
    grid = bsz // blk
    return pl.pallas_call(
        _head_body,
        grid=(grid,),
        in_specs=[
            pl.BlockSpec((blk, d), lambda i: (i, 0)),
            pl.BlockSpec((bsz, 128), lambda i: (0, 0)),
            pl.BlockSpec((blk, hlen), lambda i: (i, 0)),
            pl.BlockSpec((blk, hlen), lambda i: (i, 0)),
            pl.BlockSpec(U1.shape, lambda i: (0, 0)),
            pl.BlockSpec(ub1.shape, lambda i: (0,)),
            pl.BlockSpec(U2.shape, lambda i: (0, 0)),
            pl.BlockSpec(ub2.shape, lambda i: (0,)),
        ],
        out_specs=pl.BlockSpec((blk, bsz), lambda i: (i, 0)),
        out_shape=jax.ShapeDtypeStruct((bsz, bsz), jnp.float32),
    )(pooled, pos, ratings, mask, U1, ub1, U2, ub2)


def kernel(history_items, history_mask, history_ratings, pos_item, title_emb,
           item_feat, W1, b1, W2, b2, W3, b3, U1, ub1, U2, ub2):
    bsz, hlen = history_items.shape
    d_out = W3.shape[1]

    info = plsc.get_sparse_core_info()
    num_cores, num_subcores = info.num_cores, info.num_subcores

    E2 = _table_tower(title_emb, item_feat, W1, b1, W2, b2, W3, b3, blk=1000)

    hist_ids = history_items.reshape(-1).astype(jnp.int32)
    pos_ids = pos_item.astype(jnp.int32)
    pooled_flat, pos_rows = _sc_pool(
        E2, hist_ids, history_ratings.reshape(-1), history_mask.reshape(-1),
        pos_ids, num_cores, num_subcores, rows_per_chunk=8)

    pooled = pooled_flat.reshape(bsz, d_out)
    return _head(pooled, pos_rows, history_ratings, history_mask,
                 U1, ub1, U2, ub2, blk=256)


# speedup vs baseline: 1.3472x; 1.2556x over previous
import jax
import jax.numpy as jnp
from jax import lax
from jax.experimental import pallas as pl
from jax.experimental.pallas import tpu as pltpu
from jax.experimental.pallas import tpu_sc as plsc

_TEMP = 0.07
_LANES = 16


def _tower_body(title_ref, feat_ref, w1t_ref, w1f_ref, b1_ref, w2_ref, b2_ref,
                w3_ref, b3_ref, out_ref):
    h = jnp.dot(title_ref[...], w1t_ref[...], preferred_element_type=jnp.float32)
    h = h + jnp.dot(feat_ref[...], w1f_ref[...], preferred_element_type=jnp.float32)
    h = jax.nn.relu(h + b1_ref[...][None, :])
    h = jax.nn.relu(jnp.dot(h, w2_ref[...], preferred_element_type=jnp.float32)
                    + b2_ref[...][None, :])
    e = jnp.dot(h, w3_ref[...], preferred_element_type=jnp.float32) + b3_ref[...][None, :]
    n = jnp.sqrt(jnp.sum(e * e, axis=-1, keepdims=True))
    e = e / jnp.maximum(n, 1e-12)
    out_ref[...] = jnp.concatenate([e, jnp.zeros_like(e)], axis=-1)


def _table_tower(title_emb, item_feat, W1, b1, W2, b2, W3, b3, blk):
    v, title_d = title_emb.shape
    feat_d = item_feat.shape[1]
    d_out = W3.shape[1]
    assert v % blk == 0
    grid = v // blk
    w1t = W1[:title_d]
    w1f = W1[title_d:]
    return pl.pallas_call(
        _tower_body,
        grid=(grid,),
        in_specs=[
            pl.BlockSpec((blk, title_d), lambda i: (i, 0)),
            pl.BlockSpec((blk, feat_d), lambda i: (i, 0)),
            pl.BlockSpec(w1t.shape, lambda i: (0, 0)),
            pl.BlockSpec(w1f.shape, lambda i: (0, 0)),
            pl.BlockSpec(b1.shape, lambda i: (0,)),
            pl.BlockSpec(W2.shape, lambda i: (0, 0)),
            pl.BlockSpec(b2.shape, lambda i: (0,)),
            pl.BlockSpec(W3.shape, lambda i: (0, 0)),
            pl.BlockSpec(b3.shape, lambda i: (0,)),
        ],
        out_specs=pl.BlockSpec((blk, 2 * d_out), lambda i: (i, 0)),
        out_shape=jax.ShapeDtypeStruct((v, 2 * d_out), jnp.float32),
    )(title_emb, item_feat, w1t, w1f, b1, W2, b2, W3, b3)


def _splat(vec, lane):
    dnums = lax.GatherDimensionNumbers(
        offset_dims=(), collapsed_slice_dims=(0,), start_index_map=(0,))
    idx = jnp.full((_LANES, 1), lane, jnp.int32)
    return lax.gather(vec, idx, dnums, (1,),
                      mode=lax.GatherScatterMode.PROMISE_IN_BOUNDS)


def _sc_pool(table2, hist_ids, w_flat, pos_ids,
             num_cores, num_subcores, rows_per_chunk):
    bl = hist_ids.shape[0]
    b = pos_ids.shape[0]
    hlen = bl // b
    nw = num_cores * num_subcores
    assert b % nw == 0
    b_per_w = b // nw
    rc = rows_per_chunk
    assert b_per_w % rc == 0
    n_chunks = b_per_w // rc
    assert n_chunks % 2 == 0 and n_chunks >= 4
    wpc = rc * hlen
    assert wpc % _LANES == 0
    assert wpc % 8 == 0 and b_per_w % 8 == 0

    def body(table_hbm, ids_hbm, w_hbm, pos_hbm, pool_out, pos_out,
             ids_all, w_all, rows_v, pool_v, posi_v, sem, psem):
        wid = lax.axis_index("s") * num_cores + lax.axis_index("c")
        row0 = wid * b_per_w
        pltpu.sync_copy(pos_hbm.at[pl.ds(row0, b_per_w)], posi_v)
        pltpu.async_copy(table_hbm.at[posi_v],
                         rows_v.at[0, pl.ds(0, b_per_w)], psem).wait()
        pltpu.sync_copy(rows_v.at[0, pl.ds(0, b_per_w)],
                        pos_out.at[pl.ds(row0, b_per_w)])
        pltpu.sync_copy(ids_hbm.at[pl.ds(row0 * hlen, b_per_w * hlen)], ids_all)
        pltpu.sync_copy(w_hbm.at[pl.ds(row0 * hlen, b_per_w * hlen)], w_all)

        def start_chunk(c, p):
            pltpu.async_copy(
                table_hbm.at[ids_all.at[pl.ds(c * wpc, wpc)]],
                rows_v.at[p], sem.at[p])

        def compute_chunk(c, p):
            base = c * wpc
            for j in range(rc):
                k0 = (j * hlen) // _LANES
                k1 = (j * hlen + hlen - 1) // _LANES
                wv = [w_all[pl.ds(base + k * _LANES, _LANES)]
                      for k in range(k0, k1 + 1)]
                acc = [jnp.zeros((_LANES,), jnp.float32) for _ in range(4)]
                for l in range(hlen):
                    g = j * hlen + l
                    ws = _splat(wv[g // _LANES - k0], g % _LANES)
                    for m in range(4):
                        acc[m] = acc[m] + ws * rows_v[p, g, pl.ds(m * _LANES, _LANES)]
                for m in range(4):
                    pool_v[pl.ds(j * 64 + m * _LANES, _LANES)] = acc[m]
            pltpu.sync_copy(
                pool_v, pool_out.at[pl.ds((row0 + c * rc) * 64, rc * 64)])

        start_chunk(0, 0)
        start_chunk(1, 1)

        @pl.loop(0, n_chunks, step=2)
        def ring(t):
            for p in range(2):
                c = t + p
                pltpu.make_async_copy(
                    table_hbm.at[ids_all.at[pl.ds(c * wpc, wpc)]],
                    rows_v.at[p], sem.at[p]).wait()
                compute_chunk(c, p)

                @pl.when(c + 2 < n_chunks)
                def _():
                    start_chunk(c + 2, p)

    return pl.kernel(
        body,
        out_type=(
            jax.ShapeDtypeStruct((b * 64,), jnp.float32),
            jax.ShapeDtypeStruct((b, 128), jnp.float32),
        ),
        mesh=plsc.VectorSubcoreMesh(core_axis_name="c", subcore_axis_name="s",
                                    num_cores=num_cores,
                                    num_subcores=num_subcores),
        scratch_types=[
            pltpu.VMEM((b_per_w * hlen,), jnp.int32),
            pltpu.VMEM((b_per_w * hlen,), jnp.float32),
            pltpu.VMEM((2, wpc, 128), jnp.float32),
            pltpu.VMEM((rc * 64,), jnp.float32),
            pltpu.VMEM((b_per_w,), jnp.int32),
            pltpu.SemaphoreType.DMA((2,)),
            pltpu.SemaphoreType.DMA,
        ],
    )(table2, hist_ids, w_flat, pos_ids)


def _head_body(pooled_ref, pos_ref, r_ref, m_ref, u1_ref, ub1_ref, u2_ref,
               ub2_ref, out_ref):
    w = r_ref[...] * m_ref[...]
    s = jnp.sum(w, axis=1, keepdims=True) + 1e-8
    pooled = pooled_ref[...] / s
    h = jax.nn.relu(jnp.dot(pooled, u1_ref[...], preferred_element_type=jnp.float32)
                    + ub1_ref[...][None, :])
    user = jnp.dot(h, u2_ref[...], preferred_element_type=jnp.float32) + ub2_ref[...][None, :]
    n = jnp.sqrt(jnp.sum(user * user, axis=-1, keepdims=True))
    user = user / jnp.maximum(n, 1e-12)
    pos = pos_ref[...][:, :64]
    out_ref[...] = lax.dot_general(
        user, pos, (((1,), (1,)), ((), ())),
        preferred_element_type=jnp.float32) / _TEMP


def _head(pooled, pos, ratings, mask, U1, ub1, U2, ub2, blk):
    bsz, d = pooled.shape
    hlen = ratings.shape[1]
    assert bsz % blk == 0
    grid = bsz // blk
    return pl.pallas_call(
        _head_body,
        grid=(grid,),
        in_specs=[
            pl.BlockSpec((blk, d), lambda i: (i, 0)),
            pl.BlockSpec((bsz, 128), lambda i: (0, 0)),
            pl.BlockSpec((blk, hlen), lambda i: (i, 0)),
            pl.BlockSpec((blk, hlen), lambda i: (i, 0)),
            pl.BlockSpec(U1.shape, lambda i: (0, 0)),
            pl.BlockSpec(ub1.shape, lambda i: (0,)),
            pl.BlockSpec(U2.shape, lambda i: (0, 0)),
            pl.BlockSpec(ub2.shape, lambda i: (0,)),
        ],
        out_specs=pl.BlockSpec((blk, bsz), lambda i: (i, 0)),
        out_shape=jax.ShapeDtypeStruct((bsz, bsz), jnp.float32),
    )(pooled, pos, ratings, mask, U1, ub1, U2, ub2)


def kernel(history_items, history_mask, history_ratings, pos_item, title_emb,
           item_feat, W1, b1, W2, b2, W3, b3, U1, ub1, U2, ub2):
    bsz, hlen = history_items.shape
    d_out = W3.shape[1]

    info = plsc.get_sparse_core_info()
    num_cores, num_subcores = info.num_cores, info.num_subcores

    E2 = _table_tower(title_emb, item_feat, W1, b1, W2, b2, W3, b3, blk=1000)

    hist_ids = history_items.reshape(-1).astype(jnp.int32)
    pos_ids = pos_item.astype(jnp.int32)
    w_flat = (history_ratings * history_mask).reshape(-1)
    pooled_flat, pos_rows = _sc_pool(
        E2, hist_ids, w_flat, pos_ids, num_cores, num_subcores,
        rows_per_chunk=8)

    pooled = pooled_flat.reshape(bsz, d_out)
    return _head(pooled, pos_rows, history_ratings, history_mask,
                 U1, ub1, U2, ub2, blk=256)


# speedup vs baseline: 1.5472x; 1.1484x over previous
import jax
import jax.numpy as jnp
from jax import lax
from jax.experimental import pallas as pl
from jax.experimental.pallas import tpu as pltpu
from jax.experimental.pallas import tpu_sc as plsc

_TEMP = 0.07
_LANES = 16


def _tower_body(title_ref, feat_ref, w1t_ref, w1f_ref, b1_ref, w2_ref, b2_ref,
                w3_ref, b3_ref, out_ref):
    h = jnp.dot(title_ref[...], w1t_ref[...], preferred_element_type=jnp.float32)
    h = h + jnp.dot(feat_ref[...], w1f_ref[...], preferred_element_type=jnp.float32)
    h = jax.nn.relu(h + b1_ref[...][None, :])
    h = jax.nn.relu(jnp.dot(h, w2_ref[...], preferred_element_type=jnp.float32)
                    + b2_ref[...][None, :])
    e = jnp.dot(h, w3_ref[...], preferred_element_type=jnp.float32) + b3_ref[...][None, :]
    n = jnp.sqrt(jnp.sum(e * e, axis=-1, keepdims=True))
    e = e / jnp.maximum(n, 1e-12)
    out_ref[...] = jnp.concatenate([e, jnp.zeros_like(e)], axis=-1)


def _table_tower(title_emb, item_feat, W1, b1, W2, b2, W3, b3, blk):
    v, title_d = title_emb.shape
    feat_d = item_feat.shape[1]
    d_out = W3.shape[1]
    assert v % blk == 0
    grid = v // blk
    w1t = W1[:title_d]
    w1f = W1[title_d:]
    return pl.pallas_call(
        _tower_body,
        grid=(grid,),
        in_specs=[
            pl.BlockSpec((blk, title_d), lambda i: (i, 0)),
            pl.BlockSpec((blk, feat_d), lambda i: (i, 0)),
            pl.BlockSpec(w1t.shape, lambda i: (0, 0)),
            pl.BlockSpec(w1f.shape, lambda i: (0, 0)),
            pl.BlockSpec(b1.shape, lambda i: (0,)),
            pl.BlockSpec(W2.shape, lambda i: (0, 0)),
            pl.BlockSpec(b2.shape, lambda i: (0,)),
            pl.BlockSpec(W3.shape, lambda i: (0, 0)),
            pl.BlockSpec(b3.shape, lambda i: (0,)),
        ],
        out_specs=pl.BlockSpec((blk, 2 * d_out), lambda i: (i, 0)),
        out_shape=jax.ShapeDtypeStruct((v, 2 * d_out), jnp.float32),
    )(title_emb, item_feat, w1t, w1f, b1, W2, b2, W3, b3)


def _splat(vec, lane):
    dnums = lax.GatherDimensionNumbers(
        offset_dims=(), collapsed_slice_dims=(0,), start_index_map=(0,))
    idx = jnp.full((_LANES, 1), lane, jnp.int32)
    return lax.gather(vec, idx, dnums, (1,),
                      mode=lax.GatherScatterMode.PROMISE_IN_BOUNDS)


def _sc_pool(table2, hist_ids, w_flat, pos_ids,
             num_cores, num_subcores, rows_per_chunk):
    bl = hist_ids.shape[0]
    b = pos_ids.shape[0]
    hlen = bl // b
    nw = num_cores * num_subcores
    assert b % nw == 0
    b_per_w = b // nw
    rc = rows_per_chunk
    assert b_per_w % rc == 0
    n_chunks = b_per_w // rc
    assert n_chunks % 2 == 0 and n_chunks >= 4
    wpc = rc * hlen
    assert wpc % _LANES == 0
    assert wpc % 8 == 0 and b_per_w % 8 == 0

    def body(table_hbm, ids_hbm, w_hbm, pos_hbm, pool_out, pos_out,
             ids_all, w_all, rows_v, pool_v, posi_v, sem, psem):
        wid = lax.axis_index("s") * num_cores + lax.axis_index("c")
        row0 = wid * b_per_w
        pltpu.sync_copy(pos_hbm.at[pl.ds(row0, b_per_w)], posi_v)
        pltpu.async_copy(table_hbm.at[posi_v],
                         rows_v.at[0, pl.ds(0, b_per_w)], psem).wait()
        pltpu.sync_copy(rows_v.at[0, pl.ds(0, b_per_w)],
                        pos_out.at[pl.ds(row0, b_per_w)])
        pltpu.sync_copy(ids_hbm.at[pl.ds(row0 * hlen, b_per_w * hlen)], ids_all)
        pltpu.sync_copy(w_hbm.at[pl.ds(row0 * hlen, b_per_w * hlen)], w_all)

        def start_chunk(c, p):
            pltpu.async_copy(
                table_hbm.at[ids_all.at[pl.ds(c * wpc, wpc)]],
                rows_v.at[p], sem.at[p])

        def compute_chunk(c, p):
            base = c * wpc
            for j in range(rc):
                k0 = (j * hlen) // _LANES
                k1 = (j * hlen + hlen - 1) // _LANES
                wv = [w_all[pl.ds(base + k * _LANES, _LANES)]
                      for k in range(k0, k1 + 1)]
                acc = [jnp.zeros((_LANES,), jnp.float32) for _ in range(4)]
                for l in range(hlen):
                    g = j * hlen + l
                    ws = _splat(wv[g // _LANES - k0], g % _LANES)
                    for m in range(4):
                        acc[m] = acc[m] + ws * rows_v[p, g, pl.ds(m * _LANES, _LANES)]
                for m in range(4):
                    pool_v[pl.ds(j * 64 + m * _LANES, _LANES)] = acc[m]
            pltpu.sync_copy(
                pool_v, pool_out.at[pl.ds((row0 + c * rc) * 64, rc * 64)])

        start_chunk(0, 0)
        start_chunk(1, 1)

        @pl.loop(0, n_chunks, step=2)
        def ring(t):
            for p in range(2):
                c = t + p
                pltpu.make_async_copy(
                    table_hbm.at[ids_all.at[pl.ds(c * wpc, wpc)]],
                    rows_v.at[p], sem.at[p]).wait()
                compute_chunk(c, p)

                @pl.when(c + 2 < n_chunks)
                def _():
                    start_chunk(c + 2, p)

    return pl.kernel(
        body,
        out_type=(
            jax.ShapeDtypeStruct((b * 64,), jnp.float32),
            jax.ShapeDtypeStruct((b, 128), jnp.float32),
        ),
        mesh=plsc.VectorSubcoreMesh(core_axis_name="c", subcore_axis_name="s",
                                    num_cores=num_cores,
                                    num_subcores=num_subcores),
        scratch_types=[
            pltpu.VMEM((b_per_w * hlen,), jnp.int32),
            pltpu.VMEM((b_per_w * hlen,), jnp.float32),
            pltpu.VMEM((2, wpc, 128), jnp.float32),
            pltpu.VMEM((rc * 64,), jnp.float32),
            pltpu.VMEM((b_per_w,), jnp.int32),
            pltpu.SemaphoreType.DMA((2,)),
            pltpu.SemaphoreType.DMA,
        ],
    )(table2, hist_ids, w_flat, pos_ids)


def _head_body(pooled_ref, pos_ref, r_ref, m_ref, u1_ref, ub1_ref, u2_ref,
               ub2_ref, out_ref):
    w = r_ref[...] * m_ref[...]
    s = jnp.sum(w, axis=1, keepdims=True) + 1e-8
    pooled = pooled_ref[...] / s
    h = jax.nn.relu(jnp.dot(pooled, u1_ref[...], preferred_element_type=jnp.float32)
                    + ub1_ref[...][None, :])
    user = jnp.dot(h, u2_ref[...], preferred_element_type=jnp.float32) + ub2_ref[...][None, :]
    n = jnp.sqrt(jnp.sum(user * user, axis=-1, keepdims=True))
    user = user / jnp.maximum(n, 1e-12)
    pos = pos_ref[...][:, :64]
    out_ref[...] = lax.dot_general(
        user, pos, (((1,), (1,)), ((), ())),
        preferred_element_type=jnp.float32) / _TEMP


def _head(pooled, pos, ratings, mask, U1, ub1, U2, ub2, blk):
    bsz, d = pooled.shape
    hlen = ratings.shape[1]
    assert bsz % blk == 0
    grid = bsz // blk
    return pl.pallas_call(
        _head_body,
        grid=(grid,),
        in_specs=[
            pl.BlockSpec((blk, d), lambda i: (i, 0)),
            pl.BlockSpec((bsz, 128), lambda i: (0, 0)),
            pl.BlockSpec((blk, hlen), lambda i: (i, 0)),
            pl.BlockSpec((blk, hlen), lambda i: (i, 0)),
            pl.BlockSpec(U1.shape, lambda i: (0, 0)),
            pl.BlockSpec(ub1.shape, lambda i: (0,)),
            pl.BlockSpec(U2.shape, lambda i: (0, 0)),
            pl.BlockSpec(ub2.shape, lambda i: (0,)),
        ],
        out_specs=pl.BlockSpec((blk, bsz), lambda i: (i, 0)),
        out_shape=jax.ShapeDtypeStruct((bsz, bsz), jnp.float32),
    )(pooled, pos, ratings, mask, U1, ub1, U2, ub2)


def kernel(history_items, history_mask, history_ratings, pos_item, title_emb,
           item_feat, W1, b1, W2, b2, W3, b3, U1, ub1, U2, ub2):
    bsz, hlen = history_items.shape
    d_out = W3.shape[1]

    info = plsc.get_sparse_core_info()
    num_cores, num_subcores = info.num_cores, info.num_subcores

    E2 = _table_tower(title_emb, item_feat, W1, b1, W2, b2, W3, b3, blk=2000)

    hist_ids = history_items.reshape(-1).astype(jnp.int32)
    pos_ids = pos_item.astype(jnp.int32)
    w_flat = (history_ratings * history_mask).reshape(-1)
    pooled_flat, pos_rows = _sc_pool(
        E2, hist_ids, w_flat, pos_ids, num_cores, num_subcores,
        rows_per_chunk=8)

    pooled = pooled_flat.reshape(bsz, d_out)
    return _head(pooled, pos_rows, history_ratings, history_mask,
                 U1, ub1, U2, ub2, blk=256)


# speedup vs baseline: 1.6523x; 1.0679x over previous
import jax
import jax.numpy as jnp
from jax import lax
from jax.experimental import pallas as pl
from jax.experimental.pallas import tpu as pltpu
from jax.experimental.pallas import tpu_sc as plsc

_TEMP = 0.07
_LANES = 16


def _tower_body(title_ref, feat_ref, w1t_ref, w1f_ref, b1_ref, w2_ref, b2_ref,
                w3_ref, b3_ref, out_ref):
    h = jnp.dot(title_ref[...], w1t_ref[...], preferred_element_type=jnp.float32)
    h = h + jnp.dot(feat_ref[...], w1f_ref[...], preferred_element_type=jnp.float32)
    h = jax.nn.relu(h + b1_ref[...][None, :])
    h = jax.nn.relu(jnp.dot(h, w2_ref[...], preferred_element_type=jnp.float32)
                    + b2_ref[...][None, :])
    e = jnp.dot(h, w3_ref[...], preferred_element_type=jnp.float32) + b3_ref[...][None, :]
    n = jnp.sqrt(jnp.sum(e * e, axis=-1, keepdims=True))
    e = e / jnp.maximum(n, 1e-12)
    out_ref[...] = jnp.concatenate([e, jnp.zeros_like(e)], axis=-1)


def _table_tower(title_emb, item_feat, W1, b1, W2, b2, W3, b3, blk):
    v, title_d = title_emb.shape
    feat_d = item_feat.shape[1]
    d_out = W3.shape[1]
    assert v % blk == 0
    grid = v // blk
    w1t = W1[:title_d]
    w1f = W1[title_d:]
    return pl.pallas_call(
        _tower_body,
        grid=(grid,),
        in_specs=[
            pl.BlockSpec((blk, title_d), lambda i: (i, 0)),
            pl.BlockSpec((blk, feat_d), lambda i: (i, 0)),
            pl.BlockSpec(w1t.shape, lambda i: (0, 0)),
            pl.BlockSpec(w1f.shape, lambda i: (0, 0)),
            pl.BlockSpec(b1.shape, lambda i: (0,)),
            pl.BlockSpec(W2.shape, lambda i: (0, 0)),
            pl.BlockSpec(b2.shape, lambda i: (0,)),
            pl.BlockSpec(W3.shape, lambda i: (0, 0)),
            pl.BlockSpec(b3.shape, lambda i: (0,)),
        ],
        out_specs=pl.BlockSpec((blk, 2 * d_out), lambda i: (i, 0)),
        out_shape=jax.ShapeDtypeStruct((v, 2 * d_out), jnp.float32),
    )(title_emb, item_feat, w1t, w1f, b1, W2, b2, W3, b3)


def _splat(vec, lane):
    dnums = lax.GatherDimensionNumbers(
        offset_dims=(), collapsed_slice_dims=(0,), start_index_map=(0,))
    idx = jnp.full((_LANES, 1), lane, jnp.int32)
    return lax.gather(vec, idx, dnums, (1,),
                      mode=lax.GatherScatterMode.PROMISE_IN_BOUNDS)


def _sc_pool(table2, hist_ids, w_flat, pos_ids,
             num_cores, num_subcores, rows_per_chunk):
    bl = hist_ids.shape[0]
    b = pos_ids.shape[0]
    hlen = bl // b
    nw = num_cores * num_subcores
    assert b % nw == 0
    b_per_w = b // nw
    rc = rows_per_chunk
    assert b_per_w % rc == 0
    n_chunks = b_per_w // rc
    assert n_chunks % 2 == 0 and n_chunks >= 4
    wpc = rc * hlen
    assert wpc % _LANES == 0
    assert wpc % 8 == 0 and b_per_w % 8 == 0

    def body(table_hbm, ids_hbm, w_hbm, pos_hbm, pool_out, pos_out,
             ids_all, w_all, rows_v, pool_v, posi_v, sem, psem):
        wid = lax.axis_index("s") * num_cores + lax.axis_index("c")
        row0 = wid * b_per_w
        pltpu.sync_copy(pos_hbm.at[pl.ds(row0, b_per_w)], posi_v)
        pltpu.async_copy(table_hbm.at[posi_v],
                         rows_v.at[0, pl.ds(0, b_per_w)], psem).wait()
        pltpu.sync_copy(rows_v.at[0, pl.ds(0, b_per_w)],
                        pos_out.at[pl.ds(row0, b_per_w)])
        pltpu.sync_copy(ids_hbm.at[pl.ds(row0 * hlen, b_per_w * hlen)], ids_all)
        pltpu.sync_copy(w_hbm.at[pl.ds(row0 * hlen, b_per_w * hlen)], w_all)

        def start_chunk(c, p):
            pltpu.async_copy(
                table_hbm.at[ids_all.at[pl.ds(c * wpc, wpc)]],
                rows_v.at[p], sem.at[p])

        def compute_chunk(c, p):
            base = c * wpc
            for j in range(rc):
                k0 = (j * hlen) // _LANES
                k1 = (j * hlen + hlen - 1) // _LANES
                wv = [w_all[pl.ds(base + k * _LANES, _LANES)]
                      for k in range(k0, k1 + 1)]
                acc = [jnp.zeros((_LANES,), jnp.float32) for _ in range(4)]
                for l in range(hlen):
                    g = j * hlen + l
                    ws = _splat(wv[g // _LANES - k0], g % _LANES)
                    for m in range(4):
                        acc[m] = acc[m] + ws * rows_v[p, g, pl.ds(m * _LANES, _LANES)]
                for m in range(4):
                    pool_v[pl.ds(j * 64 + m * _LANES, _LANES)] = acc[m]
            pltpu.sync_copy(
                pool_v, pool_out.at[pl.ds((row0 + c * rc) * 64, rc * 64)])

        start_chunk(0, 0)
        start_chunk(1, 1)

        @pl.loop(0, n_chunks, step=2)
        def ring(t):
            for p in range(2):
                c = t + p
                pltpu.make_async_copy(
                    table_hbm.at[ids_all.at[pl.ds(c * wpc, wpc)]],
                    rows_v.at[p], sem.at[p]).wait()
                compute_chunk(c, p)

                @pl.when(c + 2 < n_chunks)
                def _():
                    start_chunk(c + 2, p)

    return pl.kernel(
        body,
        out_type=(
            jax.ShapeDtypeStruct((b * 64,), jnp.float32),
            jax.ShapeDtypeStruct((b, 128), jnp.float32),
        ),
        mesh=plsc.VectorSubcoreMesh(core_axis_name="c", subcore_axis_name="s",
                                    num_cores=num_cores,
                                    num_subcores=num_subcores),
        scratch_types=[
            pltpu.VMEM((b_per_w * hlen,), jnp.int32),
            pltpu.VMEM((b_per_w * hlen,), jnp.float32),
            pltpu.VMEM((2, wpc, 128), jnp.float32),
            pltpu.VMEM((rc * 64,), jnp.float32),
            pltpu.VMEM((b_per_w,), jnp.int32),
            pltpu.SemaphoreType.DMA((2,)),
            pltpu.SemaphoreType.DMA,
        ],
    )(table2, hist_ids, w_flat, pos_ids)


def _head_body(pooled_ref, pos_ref, r_ref, m_ref, u1_ref, ub1_ref, u2_ref,
               ub2_ref, out_ref):
    w = r_ref[...] * m_ref[...]
    s = jnp.sum(w, axis=1, keepdims=True) + 1e-8
    pooled = pooled_ref[...] / s
    h = jax.nn.relu(jnp.dot(pooled, u1_ref[...], preferred_element_type=jnp.float32)
                    + ub1_ref[...][None, :])
    user = jnp.dot(h, u2_ref[...], preferred_element_type=jnp.float32) + ub2_ref[...][None, :]
    n = jnp.sqrt(jnp.sum(user * user, axis=-1, keepdims=True))
    user = user / jnp.maximum(n, 1e-12)
    pos = pos_ref[...][:, :64]
    out_ref[...] = lax.dot_general(
        user, pos, (((1,), (1,)), ((), ())),
        preferred_element_type=jnp.float32) / _TEMP


def _head(pooled, pos, ratings, mask, U1, ub1, U2, ub2, blk):
    bsz, d = pooled.shape
    hlen = ratings.shape[1]
    assert bsz % blk == 0
    grid = bsz // blk
    return pl.pallas_call(
        _head_body,
        grid=(grid,),
        in_specs=[
            pl.BlockSpec((blk, d), lambda i: (i, 0)),
            pl.BlockSpec((bsz, 128), lambda i: (0, 0)),
            pl.BlockSpec((blk, hlen), lambda i: (i, 0)),
            pl.BlockSpec((blk, hlen), lambda i: (i, 0)),
            pl.BlockSpec(U1.shape, lambda i: (0, 0)),
            pl.BlockSpec(ub1.shape, lambda i: (0,)),
            pl.BlockSpec(U2.shape, lambda i: (0, 0)),
            pl.BlockSpec(ub2.shape, lambda i: (0,)),
        ],
        out_specs=pl.BlockSpec((blk, bsz), lambda i: (i, 0)),
        out_shape=jax.ShapeDtypeStruct((bsz, bsz), jnp.float32),
    )(pooled, pos, ratings, mask, U1, ub1, U2, ub2)


def kernel(history_items, history_mask, history_ratings, pos_item, title_emb,
           item_feat, W1, b1, W2, b2, W3, b3, U1, ub1, U2, ub2):
    bsz, hlen = history_items.shape
    d_out = W3.shape[1]

    info = plsc.get_sparse_core_info()
    num_cores, num_subcores = info.num_cores, info.num_subcores

    E2 = _table_tower(title_emb, item_feat, W1, b1, W2, b2, W3, b3, blk=4000)

    hist_ids = history_items.reshape(-1).astype(jnp.int32)
    pos_ids = pos_item.astype(jnp.int32)
    w_flat = (history_ratings * history_mask).reshape(-1)
    pooled_flat, pos_rows = _sc_pool(
        E2, hist_ids, w_flat, pos_ids, num_cores, num_subcores,
        rows_per_chunk=8)

    pooled = pooled_flat.reshape(bsz, d_out)
    return _head(pooled, pos_rows, history_ratings, history_mask,
                 U1, ub1, U2, ub2, blk=256)


# speedup vs baseline: 1.6576x; 1.0032x over previous
import jax
import jax.numpy as jnp
from jax import lax
from jax.experimental import pallas as pl
from jax.experimental.pallas import tpu as pltpu
from jax.experimental.pallas import tpu_sc as plsc

_TEMP = 0.07
_LANES = 16


def _tower_body(title_ref, feat_ref, w1t_ref, w1f_ref, b1_ref, w2_ref, b2_ref,
                w3_ref, b3_ref, out_ref):
    h = jnp.dot(title_ref[...], w1t_ref[...], preferred_element_type=jnp.float32)
    h = h + jnp.dot(feat_ref[...], w1f_ref[...], preferred_element_type=jnp.float32)
    h = jax.nn.relu(h + b1_ref[...][None, :])
    h = jax.nn.relu(jnp.dot(h, w2_ref[...], preferred_element_type=jnp.float32)
                    + b2_ref[...][None, :])
    e = jnp.dot(h, w3_ref[...], preferred_element_type=jnp.float32) + b3_ref[...][None, :]
    n = jnp.sqrt(jnp.sum(e * e, axis=-1, keepdims=True))
    e = e / jnp.maximum(n, 1e-12)
    out_ref[...] = jnp.concatenate([e, jnp.zeros_like(e)], axis=-1)


def _table_tower(title_emb, item_feat, W1, b1, W2, b2, W3, b3, blk):
    v, title_d = title_emb.shape
    feat_d = item_feat.shape[1]
    d_out = W3.shape[1]
    assert v % blk == 0
    grid = v // blk
    w1t = W1[:title_d]
    w1f = W1[title_d:]
    return pl.pallas_call(
        _tower_body,
        grid=(grid,),
        in_specs=[
            pl.BlockSpec((blk, title_d), lambda i: (i, 0)),
            pl.BlockSpec((blk, feat_d), lambda i: (i, 0)),
            pl.BlockSpec(w1t.shape, lambda i: (0, 0)),
            pl.BlockSpec(w1f.shape, lambda i: (0, 0)),
            pl.BlockSpec(b1.shape, lambda i: (0,)),
            pl.BlockSpec(W2.shape, lambda i: (0, 0)),
            pl.BlockSpec(b2.shape, lambda i: (0,)),
            pl.BlockSpec(W3.shape, lambda i: (0, 0)),
            pl.BlockSpec(b3.shape, lambda i: (0,)),
        ],
        out_specs=pl.BlockSpec((blk, 2 * d_out), lambda i: (i, 0)),
        out_shape=jax.ShapeDtypeStruct((v, 2 * d_out), jnp.float32),
    )(title_emb, item_feat, w1t, w1f, b1, W2, b2, W3, b3)


def _splat(vec, lane):
    dnums = lax.GatherDimensionNumbers(
        offset_dims=(), collapsed_slice_dims=(0,), start_index_map=(0,))
    idx = jnp.full((_LANES, 1), lane, jnp.int32)
    return lax.gather(vec, idx, dnums, (1,),
                      mode=lax.GatherScatterMode.PROMISE_IN_BOUNDS)


def _sc_pool(table2, hist_ids, w_flat, pos_ids,
             num_cores, num_subcores, rows_per_chunk):
    bl = hist_ids.shape[0]
    b = pos_ids.shape[0]
    hlen = bl // b
    nw = num_cores * num_subcores
    assert b % nw == 0
    b_per_w = b // nw
    rc = rows_per_chunk
    assert b_per_w % rc == 0
    n_chunks = b_per_w // rc
    assert n_chunks % 2 == 0 and n_chunks >= 4
    wpc = rc * hlen
    assert wpc % _LANES == 0
    assert wpc % 8 == 0 and b_per_w % 8 == 0

    def body(table_hbm, ids_hbm, w_hbm, pos_hbm, pool_out, pos_out,
             ids_all, w_all, rows_v, pool_v, posi_v, sem, psem):
        wid = lax.axis_index("s") * num_cores + lax.axis_index("c")
        row0 = wid * b_per_w
        pltpu.sync_copy(pos_hbm.at[pl.ds(row0, b_per_w)], posi_v)
        pltpu.async_copy(table_hbm.at[posi_v],
                         rows_v.at[0, pl.ds(0, b_per_w)], psem).wait()
        pltpu.sync_copy(rows_v.at[0, pl.ds(0, b_per_w)],
                        pos_out.at[pl.ds(row0, b_per_w)])
        pltpu.sync_copy(ids_hbm.at[pl.ds(row0 * hlen, b_per_w * hlen)], ids_all)
        pltpu.sync_copy(w_hbm.at[pl.ds(row0 * hlen, b_per_w * hlen)], w_all)

        def start_chunk(c, p):
            pltpu.async_copy(
                table_hbm.at[ids_all.at[pl.ds(c * wpc, wpc)]],
                rows_v.at[p], sem.at[p])

        def compute_chunk(c, p):
            base = c * wpc
            for j in range(rc):
                k0 = (j * hlen) // _LANES
                k1 = (j * hlen + hlen - 1) // _LANES
                wv = [w_all[pl.ds(base + k * _LANES, _LANES)]
                      for k in range(k0, k1 + 1)]
                acc = [jnp.zeros((_LANES,), jnp.float32) for _ in range(4)]
                for l in range(hlen):
                    g = j * hlen + l
                    ws = _splat(wv[g // _LANES - k0], g % _LANES)
                    for m in range(4):
                        acc[m] = acc[m] + ws * rows_v[p, g, pl.ds(m * _LANES, _LANES)]
                for m in range(4):
                    pool_v[pl.ds(j * 64 + m * _LANES, _LANES)] = acc[m]
            pltpu.sync_copy(
                pool_v, pool_out.at[pl.ds((row0 + c * rc) * 64, rc * 64)])

        start_chunk(0, 0)
        start_chunk(1, 1)

        @pl.loop(0, n_chunks, step=2)
        def ring(t):
            for p in range(2):
                c = t + p
                pltpu.make_async_copy(
                    table_hbm.at[ids_all.at[pl.ds(c * wpc, wpc)]],
                    rows_v.at[p], sem.at[p]).wait()
                compute_chunk(c, p)

                @pl.when(c + 2 < n_chunks)
                def _():
                    start_chunk(c + 2, p)

    return pl.kernel(
        body,
        out_type=(
            jax.ShapeDtypeStruct((b * 64,), jnp.float32),
            jax.ShapeDtypeStruct((b, 128), jnp.float32),
        ),
        mesh=plsc.VectorSubcoreMesh(core_axis_name="c", subcore_axis_name="s",
                                    num_cores=num_cores,
                                    num_subcores=num_subcores),
        scratch_types=[
            pltpu.VMEM((b_per_w * hlen,), jnp.int32),
            pltpu.VMEM((b_per_w * hlen,), jnp.float32),
            pltpu.VMEM((2, wpc, 128), jnp.float32),
            pltpu.VMEM((rc * 64,), jnp.float32),
            pltpu.VMEM((b_per_w,), jnp.int32),
            pltpu.SemaphoreType.DMA((2,)),
            pltpu.SemaphoreType.DMA,
        ],
    )(table2, hist_ids, w_flat, pos_ids)


def _head_body(pooled_ref, pos_ref, r_ref, m_ref, u1_ref, ub1_ref, u2_ref,
               ub2_ref, out_ref):
    w = r_ref[...] * m_ref[...]
    s = jnp.sum(w, axis=1, keepdims=True) + 1e-8
    pooled = pooled_ref[...] / s
    h = jax.nn.relu(jnp.dot(pooled, u1_ref[...], preferred_element_type=jnp.float32)
                    + ub1_ref[...][None, :])
    user = jnp.dot(h, u2_ref[...], preferred_element_type=jnp.float32) + ub2_ref[...][None, :]
    n = jnp.sqrt(jnp.sum(user * user, axis=-1, keepdims=True))
    user = user / jnp.maximum(n, 1e-12)
    pos = pos_ref[...][:, :64]
    out_ref[...] = lax.dot_general(
        user, pos, (((1,), (1,)), ((), ())),
        preferred_element_type=jnp.float32) / _TEMP


def _head(pooled, pos, ratings, mask, U1, ub1, U2, ub2, blk):
    bsz, d = pooled.shape
    hlen = ratings.shape[1]
    assert bsz % blk == 0
    grid = bsz // blk
    return pl.pallas_call(
        _head_body,
        grid=(grid,),
        in_specs=[
            pl.BlockSpec((blk, d), lambda i: (i, 0)),
            pl.BlockSpec((bsz, 128), lambda i: (0, 0)),
            pl.BlockSpec((blk, hlen), lambda i: (i, 0)),
            pl.BlockSpec((blk, hlen), lambda i: (i, 0)),
            pl.BlockSpec(U1.shape, lambda i: (0, 0)),
            pl.BlockSpec(ub1.shape, lambda i: (0,)),
            pl.BlockSpec(U2.shape, lambda i: (0, 0)),
            pl.BlockSpec(ub2.shape, lambda i: (0,)),
        ],
        out_specs=pl.BlockSpec((blk, bsz), lambda i: (i, 0)),
        out_shape=jax.ShapeDtypeStruct((bsz, bsz), jnp.float32),
    )(pooled, pos, ratings, mask, U1, ub1, U2, ub2)


def kernel(history_items, history_mask, history_ratings, pos_item, title_emb,
           item_feat, W1, b1, W2, b2, W3, b3, U1, ub1, U2, ub2):
    bsz, hlen = history_items.shape
    d_out = W3.shape[1]

    info = plsc.get_sparse_core_info()
    num_cores, num_subcores = info.num_cores, info.num_subcores

    E2 = _table_tower(title_emb, item_feat, W1, b1, W2, b2, W3, b3, blk=5000)

    hist_ids = history_items.reshape(-1).astype(jnp.int32)
    pos_ids = pos_item.astype(jnp.int32)
    w_flat = (history_ratings * history_mask).reshape(-1)
    pooled_flat, pos_rows = _sc_pool(
        E2, hist_ids, w_flat, pos_ids, num_cores, num_subcores,
        rows_per_chunk=8)

    pooled = pooled_flat.reshape(bsz, d_out)
    return _head(pooled, pos_rows, history_ratings, history_mask,
                 U1, ub1, U2, ub2, blk=512)
